# Initial kernel scaffold; baseline (speedup 1.0000x reference)
import jax
import jax.numpy as jnp
from jax.experimental import pallas as pl


def kernel(src_embeds, dst_embeds, timestamps, last_update, events_features, time_w, time_b, idx, msg_indices):
    raise NotImplementedError("write your pallas kernel here")


# speedup vs baseline: 1.0742x; 1.0742x over previous
import functools

import jax
import jax.numpy as jnp
from jax import lax
from jax.experimental import pallas as pl
from jax.experimental.pallas import tpu as pltpu
from jax.experimental.pallas import tpu_sc as plsc

_B = 16384
_D = 128
_NC = 2
_NS = 16
_NW = _NC * _NS
_BPW = _B // _NW
_CHUNK = 128
_NCHUNK = _BPW // _CHUNK


def _sc_gather(events_features, msg_idx2, idx2, last_update):
    mesh = plsc.VectorSubcoreMesh(core_axis_name="c", subcore_axis_name="s")

    @functools.partial(
        pl.kernel,
        out_type=(
            jax.ShapeDtypeStruct((_B, 4 * _D), jnp.float32),
            jax.ShapeDtypeStruct((_B,), jnp.float32),
        ),
        mesh=mesh,
        scratch_types=[
            pltpu.VMEM((_NCHUNK, _CHUNK), jnp.int32),
            pltpu.VMEM((_NCHUNK, _CHUNK), jnp.int32),
            pltpu.VMEM((_BPW, _D), jnp.float32),
            pltpu.VMEM((_BPW,), jnp.float32),
            pltpu.SemaphoreType.DMA,
            pltpu.SemaphoreType.DMA,
        ],
    )
    def k(ev_hbm, midx_hbm, idx_hbm, lu_hbm, out_hbm, luout_hbm,
          midx_v, idx_v, rows_v, lu_v, sem_e, sem_l):
        wid = lax.axis_index("s") * _NC + lax.axis_index("c")
        base = wid * _BPW
        pltpu.sync_copy(midx_hbm.at[pl.ds(wid * _NCHUNK, _NCHUNK)], midx_v)
        pltpu.sync_copy(idx_hbm.at[pl.ds(wid * _NCHUNK, _NCHUNK)], idx_v)
        copies = []
        for j in range(_NCHUNK):
            copies.append(pltpu.async_copy(
                ev_hbm.at[midx_v.at[j]],
                rows_v.at[pl.ds(j * _CHUNK, _CHUNK)], sem_e))
            copies.append(pltpu.async_copy(
                lu_hbm.at[idx_v.at[j]],
                lu_v.at[pl.ds(j * _CHUNK, _CHUNK)], sem_l))
        for c in copies:
            c.wait()
        pltpu.sync_copy(rows_v, out_hbm.at[pl.ds(base, _BPW), pl.ds(3 * _D, _D)])
        pltpu.sync_copy(lu_v, luout_hbm.at[pl.ds(base, _BPW)])

    return k(events_features, msg_idx2, idx2, last_update)


_BM = 512


def _tc_body(src_ref, dst_ref, ts_ref, lu_ref, w_ref, b_ref, _outal_ref, out_ref):
    dt = ts_ref[...] - lu_ref[...]
    out_ref[:, 0:_D] = src_ref[...]
    out_ref[:, _D:2 * _D] = dst_ref[...]
    out_ref[:, 2 * _D:3 * _D] = jnp.cos(dt * w_ref[...] + b_ref[...])


def _tc_dense(src, dst, ts2, lu2, w2, b2, out_partial):
    return pl.pallas_call(
        _tc_body,
        out_shape=jax.ShapeDtypeStruct((_B, 4 * _D), jnp.float32),
        grid=(_B // _BM,),
        in_specs=[
            pl.BlockSpec((_BM, _D), lambda i: (i, 0)),
            pl.BlockSpec((_BM, _D), lambda i: (i, 0)),
            pl.BlockSpec((_BM, 1), lambda i: (i, 0)),
            pl.BlockSpec((_BM, 1), lambda i: (i, 0)),
            pl.BlockSpec((1, _D), lambda i: (0, 0)),
            pl.BlockSpec((1, _D), lambda i: (0, 0)),
            pl.BlockSpec(memory_space=pl.ANY),
        ],
        out_specs=pl.BlockSpec((_BM, 3 * _D), lambda i: (i, 0)),
        input_output_aliases={6: 0},
        compiler_params=pltpu.CompilerParams(
            dimension_semantics=("parallel",)),
    )(src, dst, ts2, lu2, w2, b2, out_partial)


def kernel(src_embeds, dst_embeds, timestamps, last_update, events_features,
           time_w, time_b, idx, msg_indices):
    msg_idx2 = msg_indices.reshape(_B // _CHUNK, _CHUNK)
    idx2 = idx.reshape(_B // _CHUNK, _CHUNK)
    out_partial, lu = _sc_gather(events_features, msg_idx2, idx2, last_update)
    return _tc_dense(
        src_embeds, dst_embeds,
        timestamps.reshape(_B, 1), lu.reshape(_B, 1),
        time_w.reshape(1, _D), time_b.reshape(1, _D),
        out_partial)


# speedup vs baseline: 1.2073x; 1.1239x over previous
import functools

import jax
import jax.numpy as jnp
from jax import lax
from jax.experimental import pallas as pl
from jax.experimental.pallas import tpu as pltpu
from jax.experimental.pallas import tpu_sc as plsc

_B = 16384
_D = 128
_NC = 2
_NS = 16
_NW = _NC * _NS
_BPW = _B // _NW
_CHUNK = 128
_NCHUNK = _BPW // _CHUNK


def _sc_gather(events_features, msg_idx2, idx2, last_update):
    mesh = plsc.VectorSubcoreMesh(core_axis_name="c", subcore_axis_name="s")

    @functools.partial(
        pl.kernel,
        out_type=(
            jax.ShapeDtypeStruct((_B, 4 * _D), jnp.float32),
            jax.ShapeDtypeStruct((_B,), jnp.float32),
        ),
        mesh=mesh,
        scratch_types=[
            pltpu.VMEM((_NCHUNK, _CHUNK), jnp.int32),
            pltpu.VMEM((_NCHUNK, _CHUNK), jnp.int32),
            pltpu.VMEM((_BPW, _D), jnp.float32),
            pltpu.VMEM((_BPW,), jnp.float32),
            pltpu.SemaphoreType.DMA,
            pltpu.SemaphoreType.DMA,
        ],
    )
    def k(ev_hbm, midx_hbm, idx_hbm, lu_hbm, out_hbm, luout_hbm,
          midx_v, idx_v, rows_v, lu_v, sem_e, sem_l):
        wid = lax.axis_index("s") * _NC + lax.axis_index("c")
        base = wid * _BPW
        pltpu.sync_copy(midx_hbm.at[pl.ds(wid * _NCHUNK, _NCHUNK)], midx_v)
        pltpu.sync_copy(idx_hbm.at[pl.ds(wid * _NCHUNK, _NCHUNK)], idx_v)
        copies = []
        for j in range(_NCHUNK):
            copies.append(pltpu.async_copy(
                ev_hbm.at[midx_v.at[j]],
                rows_v.at[pl.ds(j * _CHUNK, _CHUNK)], sem_e))
            copies.append(pltpu.async_copy(
                lu_hbm.at[idx_v.at[j]],
                lu_v.at[pl.ds(j * _CHUNK, _CHUNK)], sem_l))
        for c in copies:
            c.wait()
        pltpu.sync_copy(rows_v, out_hbm.at[pl.ds(base, _BPW), pl.ds(3 * _D, _D)])
        pltpu.sync_copy(lu_v, luout_hbm.at[pl.ds(base, _BPW)])

    return k(events_features, msg_idx2, idx2, last_update)


_BM = 512


_INV_PI = 0.3183098861837907
_PI_HI = 3.140625
_PI_LO = 9.676535897932795e-4


def _fast_cos(x):
    n = jnp.round(x * _INV_PI)
    r = x - n * _PI_HI
    r = r - n * _PI_LO
    u = r * r
    p = 1.0 + u * (-0.5 + u * (1.0 / 24.0 + u * (-1.0 / 720.0
        + u * (1.0 / 40320.0 + u * (-1.0 / 3628800.0)))))
    nh = n * 0.5
    sign = 1.0 - 4.0 * (nh - jnp.floor(nh))
    return sign * p


def _tc_body(src_ref, dst_ref, ts_ref, lu_ref, w_ref, b_ref, _outal_ref, out_ref):
    dt = ts_ref[...] - lu_ref[...]
    out_ref[:, 0:_D] = src_ref[...]
    out_ref[:, _D:2 * _D] = dst_ref[...]
    out_ref[:, 2 * _D:3 * _D] = _fast_cos(dt * w_ref[...] + b_ref[...])


def _tc_dense(src, dst, ts2, lu2, w2, b2, out_partial):
    return pl.pallas_call(
        _tc_body,
        out_shape=jax.ShapeDtypeStruct((_B, 4 * _D), jnp.float32),
        grid=(_B // _BM,),
        in_specs=[
            pl.BlockSpec((_BM, _D), lambda i: (i, 0)),
            pl.BlockSpec((_BM, _D), lambda i: (i, 0)),
            pl.BlockSpec((_BM, 1), lambda i: (i, 0)),
            pl.BlockSpec((_BM, 1), lambda i: (i, 0)),
            pl.BlockSpec((1, _D), lambda i: (0, 0)),
            pl.BlockSpec((1, _D), lambda i: (0, 0)),
            pl.BlockSpec(memory_space=pl.ANY),
        ],
        out_specs=pl.BlockSpec((_BM, 3 * _D), lambda i: (i, 0)),
        input_output_aliases={6: 0},
        compiler_params=pltpu.CompilerParams(
            dimension_semantics=("parallel",)),
    )(src, dst, ts2, lu2, w2, b2, out_partial)


def kernel(src_embeds, dst_embeds, timestamps, last_update, events_features,
           time_w, time_b, idx, msg_indices):
    msg_idx2 = msg_indices.reshape(_B // _CHUNK, _CHUNK)
    idx2 = idx.reshape(_B // _CHUNK, _CHUNK)
    out_partial, lu = _sc_gather(events_features, msg_idx2, idx2, last_update)
    return _tc_dense(
        src_embeds, dst_embeds,
        timestamps.reshape(_B, 1), lu.reshape(_B, 1),
        time_w.reshape(1, _D), time_b.reshape(1, _D),
        out_partial)
